# baseline (speedup 1.0000x reference)
import functools
import math

import jax
import jax.numpy as jnp
from jax import lax
from jax.experimental import pallas as pl
from jax.experimental.pallas import tpu as pltpu
from jax.experimental.pallas import tpu_sc as plsc

D_MODEL = 64
SCALE = math.sqrt(D_MODEL)
LANES = 16
NC, NS = 2, 16
NW = NC * NS
BB = 128
TCH = 384

_params = pltpu.CompilerParams(
    use_tc_tiling_on_sc=True, needs_layout_passes=False
)


def _make_transpose_kernel(vocab: int):
    nch = vocab // TCH
    tail = vocab - nch * TCH
    assert tail % 2 == 0 and tail < TCH
    per_w = nch // NW
    rem = nch - per_w * NW
    mesh = plsc.VectorSubcoreMesh(core_axis_name="c", subcore_axis_name="s")

    @functools.partial(
        pl.kernel,
        out_type=jax.ShapeDtypeStruct((vocab // 2, 2 * D_MODEL), jnp.float32),
        mesh=mesh,
        scratch_types=[
            pltpu.VMEM((2, D_MODEL, TCH), jnp.float32),
            pltpu.VMEM((2, TCH // 2, 2 * D_MODEL), jnp.float32),
            pltpu.VMEM((D_MODEL, 64), jnp.float32),
            pltpu.VMEM((32, 2 * D_MODEL), jnp.float32),
            pltpu.SemaphoreType.DMA,
            pltpu.SemaphoreType.DMA,
            pltpu.SemaphoreType.DMA,
            pltpu.SemaphoreType.DMA,
        ],
        compiler_params=_params,
    )
    def tr_kernel(tt_hbm, out_hbm, inv, outv, tin, tout, is0, is1, os0, os1):
        wid = lax.axis_index("s") * NC + lax.axis_index("c")
        isems = (is0, is1)
        osems = (os0, os1)
        iota = lax.iota(jnp.int32, LANES)

        def fire(c, slot):
            pltpu.async_copy(
                tt_hbm.at[:, pl.ds(c * TCH, TCH)], inv.at[slot], isems[slot]
            )

        def drain(c, slot):
            pltpu.make_async_copy(
                tt_hbm.at[:, pl.ds(c * TCH, TCH)], inv.at[slot], isems[slot]
            ).wait()

        def owait(c, slot):
            pltpu.make_async_copy(
                outv.at[slot],
                out_hbm.at[pl.ds(c * (TCH // 2), TCH // 2)],
                osems[slot],
            ).wait()

        def chunk_of(i):
            return i * NW + wid

        nmine = per_w + 1
        fire(chunk_of(0), 0)

        @pl.loop(0, nmine)
        def c_loop(i):
            have = jnp.logical_or(i < per_w, wid < rem)

            @pl.when(have)
            def _do():
                s = lax.rem(i, 2)
                c = chunk_of(i)

                @pl.when(jnp.logical_and(i + 1 < per_w + 1,
                                         jnp.logical_or(i + 1 < per_w,
                                                        wid < rem)))
                def _start_next():
                    for t in range(2):
                        @pl.when(lax.rem(i + 1, 2) == t)
                        def _f():
                            fire(chunk_of(i + 1), t)

                for t in range(2):
                    @pl.when(s == t)
                    def _body():
                        drain(c, t)

                        @pl.when(i >= 2)
                        def _wp():
                            owait(chunk_of(i - 2), t)

                        inv2 = inv.at[t]
                        outv2 = outv.at[t]

                        @plsc.parallel_loop(0, LANES, 1, unroll=8)
                        def _j(j):
                            rotj = (iota + j) & (LANES - 1)
                            for db in range(D_MODEL // LANES):
                                dvec = rotj + db * LANES
                                for cb in range(TCH // LANES):
                                    cvec = iota + cb * LANES
                                    v = plsc.load_gather(
                                        inv2, [dvec, cvec]
                                    )
                                    plsc.store_scatter(
                                        outv2,
                                        [cvec >> 1,
                                         ((cvec & 1) << 6) + dvec],
                                        v,
                                    )

                        pltpu.async_copy(
                            outv.at[t],
                            out_hbm.at[pl.ds(c * (TCH // 2), TCH // 2)],
                            osems[t],
                        )

        last = per_w + jnp.where(wid < rem, 1, 0)
        for t in range(2):
            @pl.when(last >= 2)
            def _dr():
                @pl.when(lax.rem(last - 2 + t, 2) == t)
                def _dr2():
                    owait(chunk_of(last - 2 + t), t)

        if tail:
            assert tail == 64
            @pl.when(wid == 0)
            def _tail():
                base = nch * TCH
                pltpu.sync_copy(tt_hbm.at[:, pl.ds(base, tail)], tin)

                @pl.loop(0, D_MODEL)
                def _d(d):
                    @pl.loop(0, tail // LANES)
                    def _g(k):
                        cc = iota + k * LANES
                        v = tin[d, pl.ds(k * LANES, LANES)]
                        plsc.store_scatter(
                            tout,
                            [cc >> 1, ((cc & 1) << 6) + d],
                            v,
                        )

                pltpu.sync_copy(
                    tout, out_hbm.at[pl.ds(base // 2, tail // 2)]
                )

    return tr_kernel


def _make_gather_kernel(bsz: int, seq: int, vocab2: int):
    assert bsz == NW * BB and seq % 2 == 0
    mesh = plsc.VectorSubcoreMesh(core_axis_name="c", subcore_axis_name="s")

    @functools.partial(
        pl.kernel,
        out_type=jax.ShapeDtypeStruct((seq, D_MODEL, bsz), jnp.float32),
        mesh=mesh,
        scratch_types=[
            pltpu.VMEM((BB, seq), jnp.int32),
            pltpu.VMEM((seq, BB), jnp.int32),
            pltpu.VMEM((2, BB, 128), jnp.float32),
            pltpu.VMEM((2, D_MODEL, BB), jnp.float32),
            pltpu.SemaphoreType.DMA,
            pltpu.SemaphoreType.DMA,
            pltpu.SemaphoreType.DMA,
            pltpu.SemaphoreType.DMA,
        ],
        compiler_params=_params,
    )
    def emb_kernel(x_hbm, tab2_hbm, out_hbm, xv, idxt, gbuf, obuf,
                   gsem0, gsem1, osem0, osem1):
        wid = lax.axis_index("s") * NC + lax.axis_index("c")
        b0 = wid * BB
        gsems = (gsem0, gsem1)
        osems = (osem0, osem1)
        iota = lax.iota(jnp.int32, LANES)
        rowvecs = tuple(iota + (k * LANES) for k in range(BB // LANES))

        pltpu.sync_copy(x_hbm.at[pl.ds(b0, BB)], xv)

        @pl.loop(0, seq)
        def _build(l):
            lvec = jnp.full((LANES,), 0, jnp.int32) + l
            for k in range(BB // LANES):
                v = plsc.load_gather(xv, [rowvecs[k], lvec])
                idxt[l, pl.ds(k * LANES, LANES)] = v >> 1

        def fire(l, slot):
            pltpu.async_copy(
                tab2_hbm.at[idxt.at[l]], gbuf.at[slot], gsems[slot]
            )

        def drain(l, slot):
            pltpu.make_async_copy(
                tab2_hbm.at[idxt.at[l]], gbuf.at[slot], gsems[slot]
            ).wait()

        def owait(l, slot):
            pltpu.make_async_copy(
                obuf.at[slot],
                out_hbm.at[l, :, pl.ds(b0, BB)],
                osems[slot],
            ).wait()

        fire(0, 0)

        @pl.loop(0, seq, step=2)
        def l_loop(g):
            for s in range(2):
                l = g + s

                @pl.when(l + 1 < seq)
                def _start_next():
                    fire(l + 1, 1 - s)

                drain(l, s)

                @pl.when(l >= 2)
                def _wait_prev_out():
                    owait(l - 2, s)

                g2 = gbuf.at[s]
                o2 = obuf.at[s]
                lvec = jnp.full((LANES,), 0, jnp.int32) + l
                cols = tuple(
                    (plsc.load_gather(xv, [rowvecs[k], lvec]) & 1) * D_MODEL
                    for k in range(BB // LANES)
                )

                @plsc.parallel_loop(0, LANES, 1, unroll=8, carry=cols)
                def _j(j, carry):
                    rotj = (iota + j) & (LANES - 1)
                    for db in range(D_MODEL // LANES):
                        dvec = rotj + db * LANES
                        for k in range(BB // LANES):
                            v = plsc.load_gather(
                                g2, [rowvecs[k], carry[k] + dvec]
                            )
                            plsc.store_scatter(
                                o2, [dvec, rowvecs[k]], v * SCALE
                            )
                    return carry

                pltpu.async_copy(
                    obuf.at[s],
                    out_hbm.at[l, :, pl.ds(b0, BB)],
                    osems[s],
                )

        owait(seq - 2, 0)
        owait(seq - 1, 1)

    return emb_kernel


def kernel(x, table):
    b, l = x.shape
    v, d = table.shape
    table_t = table.T
    table2 = _make_transpose_kernel(v)(table_t)
    out_t = _make_gather_kernel(b, l, v // 2)(x.astype(jnp.int32), table2)
    return out_t.transpose(2, 0, 1)


# speedup vs baseline: 1.5266x; 1.5266x over previous
import functools
import math

import jax
import jax.numpy as jnp
from jax import lax
from jax.experimental import pallas as pl
from jax.experimental.pallas import tpu as pltpu
from jax.experimental.pallas import tpu_sc as plsc

D_MODEL = 64
SCALE = math.sqrt(D_MODEL)
LANES = 16
NC, NS = 2, 16
NW = NC * NS
BB = 128
TCH = 384

_params = pltpu.CompilerParams(
    use_tc_tiling_on_sc=True, needs_layout_passes=False
)


def _make_transpose_kernel(vocab: int):
    nch = vocab // TCH
    tail = vocab - nch * TCH
    assert tail % 2 == 0 and tail < TCH
    per_w = nch // NW
    rem = nch - per_w * NW
    mesh = plsc.VectorSubcoreMesh(core_axis_name="c", subcore_axis_name="s")

    @functools.partial(
        pl.kernel,
        out_type=jax.ShapeDtypeStruct((vocab // 2, 2 * D_MODEL), jnp.float32),
        mesh=mesh,
        scratch_types=[
            pltpu.VMEM((2, D_MODEL, TCH), jnp.float32),
            pltpu.VMEM((2, TCH // 2, 2 * D_MODEL), jnp.float32),
            pltpu.VMEM((D_MODEL, 64), jnp.float32),
            pltpu.VMEM((32, 2 * D_MODEL), jnp.float32),
            pltpu.SemaphoreType.DMA,
            pltpu.SemaphoreType.DMA,
            pltpu.SemaphoreType.DMA,
            pltpu.SemaphoreType.DMA,
        ],
        compiler_params=_params,
    )
    def tr_kernel(tt_hbm, out_hbm, inv, outv, tin, tout, is0, is1, os0, os1):
        wid = lax.axis_index("s") * NC + lax.axis_index("c")
        isems = (is0, is1)
        osems = (os0, os1)
        iota = lax.iota(jnp.int32, LANES)

        def fire(c, slot):
            pltpu.async_copy(
                tt_hbm.at[:, pl.ds(c * TCH, TCH)], inv.at[slot], isems[slot]
            )

        def drain(c, slot):
            pltpu.make_async_copy(
                tt_hbm.at[:, pl.ds(c * TCH, TCH)], inv.at[slot], isems[slot]
            ).wait()

        def owait(c, slot):
            pltpu.make_async_copy(
                outv.at[slot],
                out_hbm.at[pl.ds(c * (TCH // 2), TCH // 2)],
                osems[slot],
            ).wait()

        def chunk_of(i):
            return i * NW + wid

        nmine = per_w + 1
        fire(chunk_of(0), 0)

        @pl.loop(0, nmine)
        def c_loop(i):
            have = jnp.logical_or(i < per_w, wid < rem)

            @pl.when(have)
            def _do():
                s = lax.rem(i, 2)
                c = chunk_of(i)

                @pl.when(jnp.logical_and(i + 1 < per_w + 1,
                                         jnp.logical_or(i + 1 < per_w,
                                                        wid < rem)))
                def _start_next():
                    for t in range(2):
                        @pl.when(lax.rem(i + 1, 2) == t)
                        def _f():
                            fire(chunk_of(i + 1), t)

                for t in range(2):
                    @pl.when(s == t)
                    def _body():
                        drain(c, t)

                        @pl.when(i >= 2)
                        def _wp():
                            owait(chunk_of(i - 2), t)

                        inv2 = inv.at[t]
                        outv2 = outv.at[t]

                        @plsc.parallel_loop(0, LANES, 1, unroll=4)
                        def _j(j):
                            rotj = (iota + j) & (LANES - 1)
                            for db in range(D_MODEL // LANES):
                                dvec = rotj + db * LANES
                                for cb in range(TCH // LANES):
                                    cvec = iota + cb * LANES
                                    v = plsc.load_gather(
                                        inv2, [dvec, cvec]
                                    )
                                    plsc.store_scatter(
                                        outv2,
                                        [cvec >> 1,
                                         ((cvec & 1) << 6) + dvec],
                                        v,
                                    )

                        pltpu.async_copy(
                            outv.at[t],
                            out_hbm.at[pl.ds(c * (TCH // 2), TCH // 2)],
                            osems[t],
                        )

        last = per_w + jnp.where(wid < rem, 1, 0)
        for t in range(2):
            @pl.when(last >= 2)
            def _dr():
                @pl.when(lax.rem(last - 2 + t, 2) == t)
                def _dr2():
                    owait(chunk_of(last - 2 + t), t)

        if tail:
            assert tail == 64
            @pl.when(wid == 0)
            def _tail():
                base = nch * TCH
                pltpu.sync_copy(tt_hbm.at[:, pl.ds(base, tail)], tin)

                @pl.loop(0, D_MODEL)
                def _d(d):
                    @pl.loop(0, tail // LANES)
                    def _g(k):
                        cc = iota + k * LANES
                        v = tin[d, pl.ds(k * LANES, LANES)]
                        plsc.store_scatter(
                            tout,
                            [cc >> 1, ((cc & 1) << 6) + d],
                            v,
                        )

                pltpu.sync_copy(
                    tout, out_hbm.at[pl.ds(base // 2, tail // 2)]
                )

    return tr_kernel


def _make_gather_kernel(bsz: int, seq: int, vocab2: int):
    assert bsz == NW * BB and seq % 2 == 0
    mesh = plsc.VectorSubcoreMesh(core_axis_name="c", subcore_axis_name="s")

    @functools.partial(
        pl.kernel,
        out_type=jax.ShapeDtypeStruct((seq, D_MODEL, bsz), jnp.float32),
        mesh=mesh,
        scratch_types=[
            pltpu.VMEM((BB, seq), jnp.int32),
            pltpu.VMEM((seq, BB), jnp.int32),
            pltpu.VMEM((2, BB, 128), jnp.float32),
            pltpu.VMEM((2, D_MODEL, BB), jnp.float32),
            pltpu.SemaphoreType.DMA,
            pltpu.SemaphoreType.DMA,
            pltpu.SemaphoreType.DMA,
            pltpu.SemaphoreType.DMA,
        ],
        compiler_params=_params,
    )
    def emb_kernel(x_hbm, tab2_hbm, out_hbm, xv, idxt, gbuf, obuf,
                   gsem0, gsem1, osem0, osem1):
        wid = lax.axis_index("s") * NC + lax.axis_index("c")
        b0 = wid * BB
        gsems = (gsem0, gsem1)
        osems = (osem0, osem1)
        iota = lax.iota(jnp.int32, LANES)
        rowvecs = tuple(iota + (k * LANES) for k in range(BB // LANES))

        pltpu.sync_copy(x_hbm.at[pl.ds(b0, BB)], xv)

        @pl.loop(0, seq)
        def _build(l):
            lvec = jnp.full((LANES,), 0, jnp.int32) + l
            for k in range(BB // LANES):
                v = plsc.load_gather(xv, [rowvecs[k], lvec])
                idxt[l, pl.ds(k * LANES, LANES)] = v >> 1

        def fire(l, slot):
            pltpu.async_copy(
                tab2_hbm.at[idxt.at[l]], gbuf.at[slot], gsems[slot]
            )

        def drain(l, slot):
            pltpu.make_async_copy(
                tab2_hbm.at[idxt.at[l]], gbuf.at[slot], gsems[slot]
            ).wait()

        def owait(l, slot):
            pltpu.make_async_copy(
                obuf.at[slot],
                out_hbm.at[l, :, pl.ds(b0, BB)],
                osems[slot],
            ).wait()

        fire(0, 0)

        @pl.loop(0, seq, step=2)
        def l_loop(g):
            for s in range(2):
                l = g + s

                @pl.when(l + 1 < seq)
                def _start_next():
                    fire(l + 1, 1 - s)

                drain(l, s)

                @pl.when(l >= 2)
                def _wait_prev_out():
                    owait(l - 2, s)

                g2 = gbuf.at[s]
                o2 = obuf.at[s]
                lvec = jnp.full((LANES,), 0, jnp.int32) + l
                cols = tuple(
                    (plsc.load_gather(xv, [rowvecs[k], lvec]) & 1) * D_MODEL
                    for k in range(BB // LANES)
                )

                @plsc.parallel_loop(0, LANES, 1, unroll=4, carry=cols)
                def _j(j, carry):
                    rotj = (iota + j) & (LANES - 1)
                    for db in range(D_MODEL // LANES):
                        dvec = rotj + db * LANES
                        for k in range(BB // LANES):
                            v = plsc.load_gather(
                                g2, [rowvecs[k], carry[k] + dvec]
                            )
                            plsc.store_scatter(
                                o2, [dvec, rowvecs[k]], v * SCALE
                            )
                    return carry

                pltpu.async_copy(
                    obuf.at[s],
                    out_hbm.at[l, :, pl.ds(b0, BB)],
                    osems[s],
                )

        owait(seq - 2, 0)
        owait(seq - 1, 1)

    return emb_kernel


def kernel(x, table):
    b, l = x.shape
    v, d = table.shape
    table_t = table.T
    table2 = _make_transpose_kernel(v)(table_t)
    out_t = _make_gather_kernel(b, l, v // 2)(x.astype(jnp.int32), table2)
    return out_t.transpose(2, 0, 1)
